# baseline (speedup 1.0000x reference)
import jax
import jax.numpy as jnp
from jax.experimental import pallas as pl
from jax.experimental.pallas import tpu as pltpu

IN_CHANNELS = 128
NUM_BASIS = 128
MAX_ELEM = 100
CUTOFF = 8.0
BLOCK_E = 2000
NBUF = 4

_STEP = CUTOFF / (IN_CHANNELS - 1)
_COEFF = -0.5 / (_STEP * _STEP)


def _silu(v):
    h = 0.5 * v
    return h + h * jnp.tanh(h)


def _body(d_ref, src_ref, tgt_ref, offs_ref, lane_ref, w1_ref, b1_ref,
          stab_ref, ttab_ref, w2_ref, b2_ref, out_hbm, scratch, sems):
    i = pl.program_id(0)
    nb = pl.num_programs(0)
    buf = jax.lax.rem(i, NBUF)

    diff = d_ref[...] - offs_ref[...]
    gauss = jnp.exp(_COEFF * diff * diff).astype(jnp.bfloat16)

    lane = lane_ref[...]
    oh_s = (lane == src_ref[...]).astype(jnp.bfloat16)
    oh_t = (lane == tgt_ref[...]).astype(jnp.bfloat16)

    acc = (jnp.dot(gauss, w1_ref[...], preferred_element_type=jnp.float32)
           + jnp.dot(oh_s, stab_ref[...], preferred_element_type=jnp.float32)
           + jnp.dot(oh_t, ttab_ref[...], preferred_element_type=jnp.float32)
           + b1_ref[...])
    x = _silu(acc).astype(jnp.bfloat16)
    y = jnp.dot(x, w2_ref[...], preferred_element_type=jnp.float32) + b2_ref[...]

    @pl.when(i >= NBUF)
    def _():
        pltpu.make_async_copy(
            scratch.at[buf],
            out_hbm.at[pl.ds((i - NBUF) * BLOCK_E, BLOCK_E), :],
            sems.at[buf]).wait()

    scratch[buf, :, :] = _silu(y)
    pltpu.make_async_copy(
        scratch.at[buf],
        out_hbm.at[pl.ds(i * BLOCK_E, BLOCK_E), :],
        sems.at[buf]).start()

    @pl.when(i == nb - 1)
    def _():
        for k in range(NBUF):
            b = jax.lax.rem(i - k + NBUF, NBUF)
            pltpu.make_async_copy(
                scratch.at[b],
                out_hbm.at[pl.ds((i - k) * BLOCK_E, BLOCK_E), :],
                sems.at[b]).wait()


@jax.jit
def kernel(edge_distance, source_element, target_element, W1, b1, src_emb,
           tgt_emb, W2, b2):
    e = edge_distance.shape[0]
    nb = e // BLOCK_E
    d2 = edge_distance.reshape(e, 1)
    s2 = source_element.astype(jnp.int32).reshape(e, 1)
    t2 = target_element.astype(jnp.int32).reshape(e, 1)
    offs = (jnp.arange(IN_CHANNELS, dtype=jnp.float32) * _STEP).reshape(1, -1)
    lane = jnp.arange(IN_CHANNELS, dtype=jnp.int32).reshape(1, -1)
    pad = ((0, IN_CHANNELS - MAX_ELEM), (0, 0))
    stab = jnp.pad(src_emb, pad).astype(jnp.bfloat16)
    ttab = jnp.pad(tgt_emb, pad).astype(jnp.bfloat16)

    row = lambda i: (i, 0)
    rep = lambda i: (0, 0)
    out = pl.pallas_call(
        _body,
        grid=(nb,),
        in_specs=[
            pl.BlockSpec((BLOCK_E, 1), row),
            pl.BlockSpec((BLOCK_E, 1), row),
            pl.BlockSpec((BLOCK_E, 1), row),
            pl.BlockSpec((1, IN_CHANNELS), rep),
            pl.BlockSpec((1, IN_CHANNELS), rep),
            pl.BlockSpec((IN_CHANNELS, NUM_BASIS), rep),
            pl.BlockSpec((1, NUM_BASIS), rep),
            pl.BlockSpec((IN_CHANNELS, NUM_BASIS), rep),
            pl.BlockSpec((IN_CHANNELS, NUM_BASIS), rep),
            pl.BlockSpec((NUM_BASIS, NUM_BASIS), rep),
            pl.BlockSpec((1, NUM_BASIS), rep),
        ],
        out_specs=pl.BlockSpec(memory_space=pl.ANY),
        out_shape=jax.ShapeDtypeStruct((e, NUM_BASIS), jnp.float32),
        scratch_shapes=[
            pltpu.VMEM((NBUF, BLOCK_E, NUM_BASIS), jnp.float32),
            pltpu.SemaphoreType.DMA((NBUF,)),
        ],
        compiler_params=pltpu.CompilerParams(
            dimension_semantics=("arbitrary",)),
    )(d2, s2, t2, offs, lane, W1.astype(jnp.bfloat16), b1.reshape(1, -1),
      stab, ttab, W2.astype(jnp.bfloat16), b2.reshape(1, -1))
    return out


# speedup vs baseline: 1.0618x; 1.0618x over previous
import jax
import jax.numpy as jnp
from jax.experimental import pallas as pl
from jax.experimental.pallas import tpu as pltpu

IN_CHANNELS = 128
NUM_BASIS = 128
MAX_ELEM = 100
CUTOFF = 8.0
SUB_E = 2000
NBUF = 4
BLOCK_E = SUB_E * NBUF

_STEP = CUTOFF / (IN_CHANNELS - 1)
_COEFF = -0.5 / (_STEP * _STEP)


def _silu(v):
    h = 0.5 * v
    return h + h * jnp.tanh(h)


def _body(d_ref, src_ref, tgt_ref, offs_ref, lane_ref, w1_ref, b1_ref,
          stab_ref, ttab_ref, w2_ref, b2_ref, out_hbm, scratch, sems):
    i = pl.program_id(0)
    nb = pl.num_programs(0)

    for k in range(NBUF):
        sl = pl.ds(k * SUB_E, SUB_E)
        diff = d_ref[sl, :] - offs_ref[...]
        gauss = jnp.exp(_COEFF * diff * diff).astype(jnp.bfloat16)

        lane = lane_ref[...]
        oh_s = (lane == src_ref[sl, :]).astype(jnp.bfloat16)
        oh_t = (lane == tgt_ref[sl, :]).astype(jnp.bfloat16)

        acc = (jnp.dot(gauss, w1_ref[...], preferred_element_type=jnp.float32)
               + jnp.dot(oh_s, stab_ref[...], preferred_element_type=jnp.float32)
               + jnp.dot(oh_t, ttab_ref[...], preferred_element_type=jnp.float32)
               + b1_ref[...])
        x = _silu(acc).astype(jnp.bfloat16)
        y = (jnp.dot(x, w2_ref[...], preferred_element_type=jnp.float32)
             + b2_ref[...])

        @pl.when(i >= 1)
        def _():
            pltpu.make_async_copy(
                scratch.at[k],
                out_hbm.at[pl.ds(((i - 1) * NBUF + k) * SUB_E, SUB_E), :],
                sems.at[k]).wait()

        scratch[k, :, :] = _silu(y)
        pltpu.make_async_copy(
            scratch.at[k],
            out_hbm.at[pl.ds((i * NBUF + k) * SUB_E, SUB_E), :],
            sems.at[k]).start()

    @pl.when(i == nb - 1)
    def _():
        for k in range(NBUF):
            pltpu.make_async_copy(
                scratch.at[k],
                out_hbm.at[pl.ds((i * NBUF + k) * SUB_E, SUB_E), :],
                sems.at[k]).wait()


@jax.jit
def kernel(edge_distance, source_element, target_element, W1, b1, src_emb,
           tgt_emb, W2, b2):
    e = edge_distance.shape[0]
    nb = e // BLOCK_E
    d2 = edge_distance.reshape(e, 1)
    s2 = source_element.astype(jnp.int32).reshape(e, 1)
    t2 = target_element.astype(jnp.int32).reshape(e, 1)
    offs = (jnp.arange(IN_CHANNELS, dtype=jnp.float32) * _STEP).reshape(1, -1)
    lane = jnp.arange(IN_CHANNELS, dtype=jnp.int32).reshape(1, -1)
    pad = ((0, IN_CHANNELS - MAX_ELEM), (0, 0))
    stab = jnp.pad(src_emb, pad).astype(jnp.bfloat16)
    ttab = jnp.pad(tgt_emb, pad).astype(jnp.bfloat16)

    row = lambda i: (i, 0)
    rep = lambda i: (0, 0)
    out = pl.pallas_call(
        _body,
        grid=(nb,),
        in_specs=[
            pl.BlockSpec((BLOCK_E, 1), row),
            pl.BlockSpec((BLOCK_E, 1), row),
            pl.BlockSpec((BLOCK_E, 1), row),
            pl.BlockSpec((1, IN_CHANNELS), rep),
            pl.BlockSpec((1, IN_CHANNELS), rep),
            pl.BlockSpec((IN_CHANNELS, NUM_BASIS), rep),
            pl.BlockSpec((1, NUM_BASIS), rep),
            pl.BlockSpec((IN_CHANNELS, NUM_BASIS), rep),
            pl.BlockSpec((IN_CHANNELS, NUM_BASIS), rep),
            pl.BlockSpec((NUM_BASIS, NUM_BASIS), rep),
            pl.BlockSpec((1, NUM_BASIS), rep),
        ],
        out_specs=pl.BlockSpec(memory_space=pl.ANY),
        out_shape=jax.ShapeDtypeStruct((e, NUM_BASIS), jnp.float32),
        scratch_shapes=[
            pltpu.VMEM((NBUF, SUB_E, NUM_BASIS), jnp.float32),
            pltpu.SemaphoreType.DMA((NBUF,)),
        ],
        compiler_params=pltpu.CompilerParams(
            dimension_semantics=("arbitrary",)),
    )(d2, s2, t2, offs, lane, W1.astype(jnp.bfloat16), b1.reshape(1, -1),
      stab, ttab, W2.astype(jnp.bfloat16), b2.reshape(1, -1))
    return out
